# baseline (speedup 1.0000x reference)
import jax
import jax.numpy as jnp
from jax import lax
from jax.experimental import pallas as pl
from jax.experimental.pallas import tpu as pltpu
from jax.experimental.pallas import tpu_sc as plsc

N = 10000
HID = 16
HEADS = 8
F = HEADS * HID
NEG = 0.2
EPS = 1e-5

NC = 2
NS = 16
NW = NC * NS
B = 80
NP = 10240
RPT = NP // NS

_mesh = plsc.VectorSubcoreMesh(core_axis_name="c", subcore_axis_name="s")
_SC_PARAMS = pltpu.CompilerParams(needs_layout_passes=False)


def _edge_body(xl, xr, src, dstg, dsts, idxf, attf,
               out, den,
               xlb, xrb, ob, exb, srcb, dgb, dsb, idxb, attv, attb,
               out_sh, den_sh, sem0, sem1):
    c = lax.axis_index("c")
    s = lax.axis_index("s")
    wid = s * NC + c
    per_tile = src.shape[0] // NW
    n_chunks = per_tile // B
    zeros16 = jnp.zeros((16,), jnp.float32)
    lanes = lax.broadcasted_iota(jnp.int32, (16,), 0)

    pltpu.sync_copy(attf, attv)
    for h in range(HEADS):
        for j in range(HID):
            cd = h * 16 + ((lanes + j) & 15)
            attb[pl.ds((h * 16 + j) * 16, 16)] = plsc.load_gather(attv, [cd])

    @pl.loop(0, B)
    def _(i):
        exb[pl.ds(i * 16, 16)] = zeros16

    @pl.loop(0, B)
    def _(i):
        for j in range(8):
            ob[i, pl.ds(j * 16, 16)] = zeros16

    d0 = s * RPT * 16
    for t in range(RPT // B):
        pltpu.sync_copy(exb, den_sh.at[pl.ds(d0 + t * B * 16, B * 16)])
    r0 = s * RPT
    for t in range((RPT + B - 1) // B):
        rem = min(B, RPT - t * B)
        pltpu.sync_copy(ob.at[pl.ds(0, rem)],
                        out_sh.at[pl.ds(r0 + t * B, rem)])
    plsc.subcore_barrier()

    @pl.loop(0, n_chunks)
    def _(k):
        base = wid * per_tile + k * B
        pltpu.sync_copy(src.at[pl.ds(base, B)], srcb)
        pltpu.sync_copy(dstg.at[pl.ds(base, B)], dgb)
        pltpu.sync_copy(dsts.at[pl.ds(base, B)], dsb)
        pltpu.sync_copy(idxf.at[pl.ds(base * 16, B * 16)], idxb)
        cp0 = pltpu.async_copy(xl.at[srcb], xlb, sem0)
        cp1 = pltpu.async_copy(xr.at[dgb], xrb, sem1)
        cp0.wait()
        cp1.wait()

        @pl.loop(0, B // 16)
        def _(g):
            eidx = g * 16 + lanes

            @pl.loop(0, HEADS)
            def _(h):
                acc = zeros16
                for j in range(HID):
                    cd = h * 16 + ((lanes + j) & 15)
                    a = plsc.load_gather(xlb, [eidx, cd])
                    bv = plsc.load_gather(xrb, [eidx, cd])
                    m = a + bv
                    m = jnp.where(m > 0, m, NEG * m)
                    acc = acc + m * attb[pl.ds((h * 16 + j) * 16, 16)]
                exv = jnp.exp(acc)
                exb[pl.ds(h * B + g * 16, 16)] = exv
                for j in range(HID):
                    cd = h * 16 + ((lanes + j) & 15)
                    a = plsc.load_gather(xlb, [eidx, cd])
                    plsc.store_scatter(ob, [eidx, cd], a * exv)

        pltpu.sync_copy(ob, out_sh.at[dsb], add=True)
        pltpu.sync_copy(exb, den_sh.at[idxb], add=True)

    plsc.subcore_barrier()
    for t in range(RPT // B):
        pltpu.sync_copy(den_sh.at[pl.ds(d0 + t * B * 16, B * 16)], exb)
        pltpu.sync_copy(exb, den.at[c, pl.ds(d0 + t * B * 16, B * 16)])
    for t in range((RPT + B - 1) // B):
        rem = min(B, RPT - t * B)
        pltpu.sync_copy(out_sh.at[pl.ds(r0 + t * B, rem)],
                        xlb.at[pl.ds(0, rem)])
        pltpu.sync_copy(xlb.at[pl.ds(0, rem)],
                        out.at[c, pl.ds(r0 + t * B, rem)])


def _make_edge(ep):
    return pl.kernel(
        _edge_body,
        out_type=[
            jax.ShapeDtypeStruct((NC, NP, F), jnp.float32),
            jax.ShapeDtypeStruct((NC, NP * 16), jnp.float32),
        ],
        mesh=_mesh,
        compiler_params=_SC_PARAMS,
        scratch_types=[
            pltpu.VMEM((B, F), jnp.float32),
            pltpu.VMEM((B, F), jnp.float32),
            pltpu.VMEM((B, F), jnp.float32),
            pltpu.VMEM((B * 16,), jnp.float32),
            pltpu.VMEM((B,), jnp.int32),
            pltpu.VMEM((B,), jnp.int32),
            pltpu.VMEM((B,), jnp.int32),
            pltpu.VMEM((B * 16,), jnp.int32),
            pltpu.VMEM((F,), jnp.float32),
            pltpu.VMEM((F * 16,), jnp.float32),
            pltpu.VMEM_SHARED((NP, F), jnp.float32),
            pltpu.VMEM_SHARED((NP * 16,), jnp.float32),
            pltpu.SemaphoreType.DMA,
            pltpu.SemaphoreType.DMA,
        ],
    )


def _mm2_body(x_ref, wl_ref, wr_ref, xl_ref, xr_ref):
    x = x_ref[...]
    xl_ref[...] = jnp.dot(x, wl_ref[...], preferred_element_type=jnp.float32)
    xr_ref[...] = jnp.dot(x, wr_ref[...], preferred_element_type=jnp.float32)


def _mid_body(o_ref, d_ref, r_ref, b0_ref, g0_ref, bb0_ref,
              wl1_ref, wr1_ref, xl1_ref, xr1_ref):
    raw = o_ref[0, pl.ds(0, N), :] + o_ref[1, pl.ds(0, N), :]
    den = d_ref[0, pl.ds(0, N), :] + d_ref[1, pl.ds(0, N), :]
    dexp = jnp.dot(den, r_ref[...], preferred_element_type=jnp.float32)
    h = raw / (dexp + 1e-16) + b0_ref[...]
    mu = jnp.mean(h, axis=0)
    xc = h - mu
    var = jnp.mean(xc * xc, axis=0)
    hn = xc * lax.rsqrt(var + EPS) * g0_ref[...] + bb0_ref[...]
    he = jnp.where(hn > 0, hn, jnp.exp(hn) - 1.0)
    xl1_ref[...] = jnp.dot(he, wl1_ref[...],
                           preferred_element_type=jnp.float32)
    xr1_ref[...] = jnp.dot(he, wr1_ref[...],
                           preferred_element_type=jnp.float32)


def _fin_body(o_ref, d_ref, r_ref, m_ref, b1_ref, g1_ref,
              bb1_ref, cw_ref, cb_ref, out_ref):
    raw = o_ref[0, pl.ds(0, N), :] + o_ref[1, pl.ds(0, N), :]
    den = d_ref[0, pl.ds(0, N), :] + d_ref[1, pl.ds(0, N), :]
    dexp = jnp.dot(den, r_ref[...], preferred_element_type=jnp.float32)
    hm = raw / (dexp + 1e-16)
    hv = jnp.dot(hm, m_ref[...], preferred_element_type=jnp.float32)
    hv = hv + b1_ref[...]
    mu = jnp.mean(hv, axis=0)
    xc = hv - mu
    var = jnp.mean(xc * xc, axis=0)
    hn = xc * lax.rsqrt(var + EPS) * g1_ref[...] + bb1_ref[...]
    out_ref[...] = jnp.dot(hn, cw_ref[...],
                           preferred_element_type=jnp.float32) + cb_ref[...]


@jax.jit
def kernel(x, edge_index, conv0_Wl, conv0_Wr, conv0_att, conv0_b, bn0_g,
           bn0_b, conv1_Wl, conv1_Wr, conv1_att, conv1_b, bn1_g, bn1_b,
           cls_W, cls_b):
    e = edge_index.shape[1]
    et = e + N
    ep = ((et + NW * B - 1) // (NW * B)) * (NW * B)
    pad = ep - et

    ei = edge_index.astype(jnp.int32)
    loops = jnp.arange(N, dtype=jnp.int32)
    zpad = jnp.zeros((pad,), jnp.int32)
    src = jnp.concatenate([ei[0], loops, zpad])
    dstg = jnp.concatenate([ei[1], loops, zpad])
    dsts = jnp.concatenate([ei[1], loops, jnp.full((pad,), N, jnp.int32)])
    idxf = (dsts.reshape(-1, 1, B) * 16
            + jnp.arange(16, dtype=jnp.int32).reshape(1, 16, 1)).reshape(-1)

    mm2 = pl.pallas_call(
        _mm2_body,
        out_shape=[jax.ShapeDtypeStruct((N, F), jnp.float32)] * 2,
    )
    edge = _make_edge(ep)

    rmat = jnp.zeros((16, F), jnp.float32)
    rmat = rmat.at[jnp.repeat(jnp.arange(8), 16),
                   jnp.arange(F)].set(1.0)
    mmat = jnp.tile(jnp.eye(HID, dtype=jnp.float32), (HEADS, 1)) / HEADS

    xl0, xr0 = mm2(x, conv0_Wl, conv0_Wr)
    o0, den0 = edge(xl0, xr0, src, dstg, dsts, idxf, conv0_att.reshape(-1))

    mid = pl.pallas_call(
        _mid_body,
        out_shape=[jax.ShapeDtypeStruct((N, F), jnp.float32)] * 2,
    )
    xl1, xr1 = mid(o0, den0.reshape(NC, NP, 16),
                   rmat, conv0_b, bn0_g, bn0_b, conv1_Wl, conv1_Wr)

    o1, den1 = edge(xl1, xr1, src, dstg, dsts, idxf, conv1_att.reshape(-1))

    fin = pl.pallas_call(
        _fin_body,
        out_shape=jax.ShapeDtypeStruct((N, 2), jnp.float32),
    )
    return fin(o1, den1.reshape(NC, NP, 16),
               rmat, mmat, conv1_b, bn1_g, bn1_b, cls_W, cls_b)


# speedup vs baseline: 1.0030x; 1.0030x over previous
import jax
import jax.numpy as jnp
from jax import lax
from jax.experimental import pallas as pl
from jax.experimental.pallas import tpu as pltpu
from jax.experimental.pallas import tpu_sc as plsc

N = 10000
HID = 16
HEADS = 8
F = HEADS * HID
NEG = 0.2
EPS = 1e-5

NC = 2
NS = 16
NW = NC * NS
B = 64
NP = 10240
RPT = NP // NS

_mesh = plsc.VectorSubcoreMesh(core_axis_name="c", subcore_axis_name="s")
_SC_PARAMS = pltpu.CompilerParams(needs_layout_passes=False)


def _edge_body(xl, xr, src, dstg, dsts, idxf, attf,
               out, den,
               xlb, xrb, oba, obb, exba, exbb, srcb, dgb, dsba, dsbb,
               idxba, idxbb, attb,
               out_sh, den_sh, sem0, sem1, ssoa, ssob, ssda, ssdb):
    c = lax.axis_index("c")
    s = lax.axis_index("s")
    wid = s * NC + c
    per_tile = src.shape[0] // NW
    n_chunks = per_tile // B
    zeros16 = jnp.zeros((16,), jnp.float32)
    lanes = lax.broadcasted_iota(jnp.int32, (16,), 0)

    pltpu.sync_copy(attf, attb.at[pl.ds(0, F)])
    for h in reversed(range(HEADS)):
        for j in reversed(range(HID)):
            cd = h * 16 + ((lanes + j) & 15)
            attb[pl.ds((h * 16 + j) * 16, 16)] = plsc.load_gather(
                attb.at[pl.ds(0, F)], [cd])

    @pl.loop(0, B // 2)
    def _(i):
        exba[pl.ds(i * 16, 16)] = zeros16

    @pl.loop(0, B)
    def _(i):
        for j in range(8):
            oba[i, pl.ds(j * 16, 16)] = zeros16

    d0 = s * RPT * 8
    for t in range(RPT * 8 // (B * 8)):
        pltpu.sync_copy(exba, den_sh.at[pl.ds(d0 + t * B * 8, B * 8)])
    r0 = s * RPT
    for t in range(RPT // B):
        pltpu.sync_copy(oba, out_sh.at[pl.ds(r0 + t * B, B)])

    base0 = wid * per_tile
    pltpu.sync_copy(src.at[pl.ds(base0, B)], srcb)
    pltpu.sync_copy(dstg.at[pl.ds(base0, B)], dgb)
    pltpu.sync_copy(dsts.at[pl.ds(base0, B)], dsba)
    pltpu.sync_copy(idxf.at[pl.ds(base0 * 8, B * 8)], idxba)
    pltpu.async_copy(xl.at[srcb], xlb, sem0)
    pltpu.async_copy(xr.at[dgb], xrb, sem1)
    plsc.subcore_barrier()

    @pl.loop(0, n_chunks // 2)
    def _(kk):
        for p in range(2):
            k = kk * 2 + p
            ob_p, exb_p = (oba, exba) if p == 0 else (obb, exbb)
            dsb_p, idxb_p = (dsba, idxba) if p == 0 else (dsbb, idxbb)
            dsb_q, idxb_q = (dsbb, idxbb) if p == 0 else (dsba, idxba)
            sso_p, ssd_p = (ssoa, ssda) if p == 0 else (ssob, ssdb)
            sso_q, ssd_q = (ssob, ssdb) if p == 0 else (ssoa, ssda)

            pltpu.make_async_copy(xl.at[srcb], xlb, sem0).wait()
            pltpu.make_async_copy(xr.at[dgb], xrb, sem1).wait()

            @pl.loop(0, B // 16)
            def _(g):
                eidx = g * 16 + lanes

                @pl.loop(0, HEADS)
                def _(h):
                    acc = zeros16
                    for j in range(HID):
                        cd = h * 16 + ((lanes + j) & 15)
                        a = plsc.load_gather(xlb, [eidx, cd])
                        bv = plsc.load_gather(xrb, [eidx, cd])
                        m = a + bv
                        m = jnp.where(m > 0, m, NEG * m)
                        acc = acc + m * attb[pl.ds((h * 16 + j) * 16, 16)]
                    exv = jnp.exp(acc)
                    exb_p[pl.ds(h * B + g * 16, 16)] = exv
                    for j in range(HID):
                        cd = h * 16 + ((lanes + j) & 15)
                        a = plsc.load_gather(xlb, [eidx, cd])
                        plsc.store_scatter(ob_p, [eidx, cd], a * exv)

            @pl.when(k >= 1)
            def _():
                ob_q, exb_q = (obb, exbb) if p == 0 else (oba, exba)
                pltpu.make_async_copy(
                    ob_q, out_sh.at[dsb_q], sso_q).wait()
                pltpu.make_async_copy(
                    exb_q, den_sh.at[idxb_q], ssd_q).wait()

            @pl.when(k < n_chunks - 1)
            def _():
                base = wid * per_tile + (k + 1) * B
                pltpu.sync_copy(src.at[pl.ds(base, B)], srcb)
                pltpu.sync_copy(dstg.at[pl.ds(base, B)], dgb)
                pltpu.sync_copy(dsts.at[pl.ds(base, B)], dsb_q)
                pltpu.sync_copy(idxf.at[pl.ds(base * 8, B * 8)], idxb_q)
                pltpu.async_copy(xl.at[srcb], xlb, sem0)
                pltpu.async_copy(xr.at[dgb], xrb, sem1)

            pltpu.async_copy(ob_p, out_sh.at[dsb_p], sso_p, add=True)
            pltpu.async_copy(exb_p, den_sh.at[idxb_p], ssd_p, add=True)

    pltpu.make_async_copy(obb, out_sh.at[dsbb], ssob).wait()
    pltpu.make_async_copy(exbb, den_sh.at[idxbb], ssdb).wait()

    plsc.subcore_barrier()
    for t in range(RPT * 8 // (B * 8)):
        pltpu.sync_copy(den_sh.at[pl.ds(d0 + t * B * 8, B * 8)], exba)
        pltpu.sync_copy(exba, den.at[c, pl.ds(d0 + t * B * 8, B * 8)])
    for t in range(RPT // B):
        pltpu.sync_copy(out_sh.at[pl.ds(r0 + t * B, B)], xlb)
        pltpu.sync_copy(xlb, out.at[c, pl.ds(r0 + t * B, B)])


def _make_edge(ep):
    return pl.kernel(
        _edge_body,
        out_type=[
            jax.ShapeDtypeStruct((NC, NP, F), jnp.float32),
            jax.ShapeDtypeStruct((NC, NP * 8), jnp.float32),
        ],
        mesh=_mesh,
        compiler_params=_SC_PARAMS,
        scratch_types=[
            pltpu.VMEM((B, F), jnp.float32),
            pltpu.VMEM((B, F), jnp.float32),
            pltpu.VMEM((B, F), jnp.float32),
            pltpu.VMEM((B, F), jnp.float32),
            pltpu.VMEM((B * 8,), jnp.float32),
            pltpu.VMEM((B * 8,), jnp.float32),
            pltpu.VMEM((B,), jnp.int32),
            pltpu.VMEM((B,), jnp.int32),
            pltpu.VMEM((B,), jnp.int32),
            pltpu.VMEM((B,), jnp.int32),
            pltpu.VMEM((B * 8,), jnp.int32),
            pltpu.VMEM((B * 8,), jnp.int32),
            pltpu.VMEM((F * 16,), jnp.float32),
            pltpu.VMEM_SHARED((NP, F), jnp.float32),
            pltpu.VMEM_SHARED((NP * 8,), jnp.float32),
            pltpu.SemaphoreType.DMA,
            pltpu.SemaphoreType.DMA,
            pltpu.SemaphoreType.DMA,
            pltpu.SemaphoreType.DMA,
            pltpu.SemaphoreType.DMA,
            pltpu.SemaphoreType.DMA,
        ],
    )


def _mm2_body(x_ref, wl_ref, wr_ref, xl_ref, xr_ref):
    x = x_ref[...]
    xl_ref[...] = jnp.dot(x, wl_ref[...], preferred_element_type=jnp.float32)
    xr_ref[...] = jnp.dot(x, wr_ref[...], preferred_element_type=jnp.float32)


def _mid_body(o_ref, d_ref, r_ref, b0_ref, g0_ref, bb0_ref,
              wl1_ref, wr1_ref, xl1_ref, xr1_ref):
    raw = o_ref[0, pl.ds(0, N), :] + o_ref[1, pl.ds(0, N), :]
    den = d_ref[0, pl.ds(0, N), :] + d_ref[1, pl.ds(0, N), :]
    dexp = jnp.dot(den, r_ref[...], preferred_element_type=jnp.float32)
    h = raw / (dexp + 1e-16) + b0_ref[...]
    mu = jnp.mean(h, axis=0)
    xc = h - mu
    var = jnp.mean(xc * xc, axis=0)
    hn = xc * lax.rsqrt(var + EPS) * g0_ref[...] + bb0_ref[...]
    he = jnp.where(hn > 0, hn, jnp.exp(hn) - 1.0)
    xl1_ref[...] = jnp.dot(he, wl1_ref[...],
                           preferred_element_type=jnp.float32)
    xr1_ref[...] = jnp.dot(he, wr1_ref[...],
                           preferred_element_type=jnp.float32)


def _fin_body(o_ref, d_ref, r_ref, m_ref, b1_ref, g1_ref,
              bb1_ref, cw_ref, cb_ref, out_ref):
    raw = o_ref[0, pl.ds(0, N), :] + o_ref[1, pl.ds(0, N), :]
    den = d_ref[0, pl.ds(0, N), :] + d_ref[1, pl.ds(0, N), :]
    dexp = jnp.dot(den, r_ref[...], preferred_element_type=jnp.float32)
    hm = raw / (dexp + 1e-16)
    hv = jnp.dot(hm, m_ref[...], preferred_element_type=jnp.float32)
    hv = hv + b1_ref[...]
    mu = jnp.mean(hv, axis=0)
    xc = hv - mu
    var = jnp.mean(xc * xc, axis=0)
    hn = xc * lax.rsqrt(var + EPS) * g1_ref[...] + bb1_ref[...]
    out_ref[...] = jnp.dot(hn, cw_ref[...],
                           preferred_element_type=jnp.float32) + cb_ref[...]


@jax.jit
def kernel(x, edge_index, conv0_Wl, conv0_Wr, conv0_att, conv0_b, bn0_g,
           bn0_b, conv1_Wl, conv1_Wr, conv1_att, conv1_b, bn1_g, bn1_b,
           cls_W, cls_b):
    e = edge_index.shape[1]
    et = e + N
    blk = 2 * NW * B
    ep = ((et + blk - 1) // blk) * blk
    pad = ep - et

    ei = edge_index.astype(jnp.int32)
    loops = jnp.arange(N, dtype=jnp.int32)
    zpad = jnp.zeros((pad,), jnp.int32)
    src = jnp.concatenate([ei[0], loops, zpad])
    dstg = jnp.concatenate([ei[1], loops, zpad])
    dsts = jnp.concatenate([ei[1], loops, jnp.full((pad,), N, jnp.int32)])
    idxf = (dsts.reshape(-1, 1, B) * 8
            + jnp.arange(8, dtype=jnp.int32).reshape(1, 8, 1)).reshape(-1)

    mm2 = pl.pallas_call(
        _mm2_body,
        out_shape=[jax.ShapeDtypeStruct((N, F), jnp.float32)] * 2,
    )
    edge = _make_edge(ep)

    rmat = jnp.zeros((8, F), jnp.float32)
    rmat = rmat.at[jnp.repeat(jnp.arange(8), 16),
                   jnp.arange(F)].set(1.0)
    mmat = jnp.tile(jnp.eye(HID, dtype=jnp.float32), (HEADS, 1)) / HEADS

    xl0, xr0 = mm2(x, conv0_Wl, conv0_Wr)
    o0, den0 = edge(xl0, xr0, src, dstg, dsts, idxf, conv0_att.reshape(-1))

    mid = pl.pallas_call(
        _mid_body,
        out_shape=[jax.ShapeDtypeStruct((N, F), jnp.float32)] * 2,
    )
    xl1, xr1 = mid(o0, den0.reshape(NC, NP, 8),
                   rmat, conv0_b, bn0_g, bn0_b, conv1_Wl, conv1_Wr)

    o1, den1 = edge(xl1, xr1, src, dstg, dsts, idxf, conv1_att.reshape(-1))

    fin = pl.pallas_call(
        _fin_body,
        out_shape=jax.ShapeDtypeStruct((N, 2), jnp.float32),
    )
    return fin(o1, den1.reshape(NC, NP, 8),
               rmat, mmat, conv1_b, bn1_g, bn1_b, cls_W, cls_b)


# speedup vs baseline: 1.0091x; 1.0060x over previous
import jax
import jax.numpy as jnp
from jax import lax
from jax.experimental import pallas as pl
from jax.experimental.pallas import tpu as pltpu
from jax.experimental.pallas import tpu_sc as plsc

N = 10000
HID = 16
HEADS = 8
F = HEADS * HID
NEG = 0.2
EPS = 1e-5

NC = 2
NS = 16
NW = NC * NS
B = 64
NP = 10240
RPT = NP // NS

_mesh = plsc.VectorSubcoreMesh(core_axis_name="c", subcore_axis_name="s")
_SC_PARAMS = pltpu.CompilerParams(needs_layout_passes=False)


def _edge_body(xl, xr, src, dstg, dsts, idxf, attf,
               out, den,
               xlb, xrb, oba, obb, exba, exbb, srcb, dgb, dsba, dsbb,
               idxba, idxbb, attb,
               out_sh, den_sh, sem0, sem1, ssoa, ssob, ssda, ssdb):
    c = lax.axis_index("c")
    s = lax.axis_index("s")
    wid = s * NC + c
    per_tile = src.shape[0] // NW
    n_chunks = per_tile // B
    zeros16 = jnp.zeros((16,), jnp.float32)
    lanes = lax.broadcasted_iota(jnp.int32, (16,), 0)

    pltpu.sync_copy(attf, attb.at[pl.ds(0, F)])
    for h in reversed(range(HEADS)):
        for j in reversed(range(HID)):
            cd = h * 16 + ((lanes + j) & 15)
            attb[pl.ds((h * 16 + j) * 16, 16)] = plsc.load_gather(
                attb.at[pl.ds(0, F)], [cd])

    @pl.loop(0, B // 2)
    def _(i):
        exba[pl.ds(i * 16, 16)] = zeros16

    @pl.loop(0, B)
    def _(i):
        for j in range(8):
            oba[i, pl.ds(j * 16, 16)] = zeros16

    d0 = s * RPT * 8
    for t in range(RPT * 8 // (B * 8)):
        pltpu.sync_copy(exba, den_sh.at[pl.ds(d0 + t * B * 8, B * 8)])
    r0 = s * RPT
    for t in range(RPT // B):
        pltpu.sync_copy(oba, out_sh.at[pl.ds(r0 + t * B, B)])

    base0 = wid * per_tile
    pltpu.sync_copy(src.at[pl.ds(base0, B)], srcb)
    pltpu.sync_copy(dstg.at[pl.ds(base0, B)], dgb)
    pltpu.sync_copy(dsts.at[pl.ds(base0, B)], dsba)
    pltpu.sync_copy(idxf.at[pl.ds(base0 * 8, B * 8)], idxba)
    pltpu.async_copy(xl.at[srcb], xlb, sem0)
    pltpu.async_copy(xr.at[dgb], xrb, sem1)
    plsc.subcore_barrier()

    @pl.loop(0, n_chunks // 2)
    def _(kk):
        for p in range(2):
            k = kk * 2 + p
            ob_p, exb_p = (oba, exba) if p == 0 else (obb, exbb)
            dsb_p, idxb_p = (dsba, idxba) if p == 0 else (dsbb, idxbb)
            dsb_q, idxb_q = (dsbb, idxbb) if p == 0 else (dsba, idxba)
            sso_p, ssd_p = (ssoa, ssda) if p == 0 else (ssob, ssdb)
            sso_q, ssd_q = (ssob, ssdb) if p == 0 else (ssoa, ssda)

            pltpu.make_async_copy(xl.at[srcb], xlb, sem0).wait()
            pltpu.make_async_copy(xr.at[dgb], xrb, sem1).wait()

            @pl.loop(0, B // 16)
            def _(g):
                eidx = g * 16 + lanes

                for h in range(HEADS):
                    acc = zeros16
                    for j in range(HID):
                        cd = h * 16 + ((lanes + j) & 15)
                        a = plsc.load_gather(xlb, [eidx, cd])
                        bv = plsc.load_gather(xrb, [eidx, cd])
                        m = a + bv
                        m = jnp.where(m > 0, m, NEG * m)
                        acc = acc + m * attb[pl.ds((h * 16 + j) * 16, 16)]
                    exv = jnp.exp(acc)
                    exb_p[pl.ds(h * B + g * 16, 16)] = exv
                    for j in range(HID):
                        cd = h * 16 + ((lanes + j) & 15)
                        a = plsc.load_gather(xlb, [eidx, cd])
                        plsc.store_scatter(ob_p, [eidx, cd], a * exv)

            @pl.when(k >= 1)
            def _():
                ob_q, exb_q = (obb, exbb) if p == 0 else (oba, exba)
                pltpu.make_async_copy(
                    ob_q, out_sh.at[dsb_q], sso_q).wait()
                pltpu.make_async_copy(
                    exb_q, den_sh.at[idxb_q], ssd_q).wait()

            @pl.when(k < n_chunks - 1)
            def _():
                base = wid * per_tile + (k + 1) * B
                pltpu.sync_copy(src.at[pl.ds(base, B)], srcb)
                pltpu.sync_copy(dstg.at[pl.ds(base, B)], dgb)
                pltpu.sync_copy(dsts.at[pl.ds(base, B)], dsb_q)
                pltpu.sync_copy(idxf.at[pl.ds(base * 8, B * 8)], idxb_q)
                pltpu.async_copy(xl.at[srcb], xlb, sem0)
                pltpu.async_copy(xr.at[dgb], xrb, sem1)

            pltpu.async_copy(ob_p, out_sh.at[dsb_p], sso_p, add=True)
            pltpu.async_copy(exb_p, den_sh.at[idxb_p], ssd_p, add=True)

    pltpu.make_async_copy(obb, out_sh.at[dsbb], ssob).wait()
    pltpu.make_async_copy(exbb, den_sh.at[idxbb], ssdb).wait()

    plsc.subcore_barrier()
    for t in range(RPT * 8 // (B * 8)):
        pltpu.sync_copy(den_sh.at[pl.ds(d0 + t * B * 8, B * 8)], exba)
        pltpu.sync_copy(exba, den.at[c, pl.ds(d0 + t * B * 8, B * 8)])
    for t in range(RPT // B):
        pltpu.sync_copy(out_sh.at[pl.ds(r0 + t * B, B)], xlb)
        pltpu.sync_copy(xlb, out.at[c, pl.ds(r0 + t * B, B)])


def _make_edge(ep):
    return pl.kernel(
        _edge_body,
        out_type=[
            jax.ShapeDtypeStruct((NC, NP, F), jnp.float32),
            jax.ShapeDtypeStruct((NC, NP * 8), jnp.float32),
        ],
        mesh=_mesh,
        compiler_params=_SC_PARAMS,
        scratch_types=[
            pltpu.VMEM((B, F), jnp.float32),
            pltpu.VMEM((B, F), jnp.float32),
            pltpu.VMEM((B, F), jnp.float32),
            pltpu.VMEM((B, F), jnp.float32),
            pltpu.VMEM((B * 8,), jnp.float32),
            pltpu.VMEM((B * 8,), jnp.float32),
            pltpu.VMEM((B,), jnp.int32),
            pltpu.VMEM((B,), jnp.int32),
            pltpu.VMEM((B,), jnp.int32),
            pltpu.VMEM((B,), jnp.int32),
            pltpu.VMEM((B * 8,), jnp.int32),
            pltpu.VMEM((B * 8,), jnp.int32),
            pltpu.VMEM((F * 16,), jnp.float32),
            pltpu.VMEM_SHARED((NP, F), jnp.float32),
            pltpu.VMEM_SHARED((NP * 8,), jnp.float32),
            pltpu.SemaphoreType.DMA,
            pltpu.SemaphoreType.DMA,
            pltpu.SemaphoreType.DMA,
            pltpu.SemaphoreType.DMA,
            pltpu.SemaphoreType.DMA,
            pltpu.SemaphoreType.DMA,
        ],
    )


def _mm2_body(x_ref, wl_ref, wr_ref, xl_ref, xr_ref):
    x = x_ref[...]
    xl_ref[...] = jnp.dot(x, wl_ref[...], preferred_element_type=jnp.float32)
    xr_ref[...] = jnp.dot(x, wr_ref[...], preferred_element_type=jnp.float32)


def _mid_body(o_ref, d_ref, r_ref, b0_ref, g0_ref, bb0_ref,
              wl1_ref, wr1_ref, xl1_ref, xr1_ref):
    raw = o_ref[0, pl.ds(0, N), :] + o_ref[1, pl.ds(0, N), :]
    den = d_ref[0, pl.ds(0, N), :] + d_ref[1, pl.ds(0, N), :]
    dexp = jnp.dot(den, r_ref[...], preferred_element_type=jnp.float32)
    h = raw / (dexp + 1e-16) + b0_ref[...]
    mu = jnp.mean(h, axis=0)
    xc = h - mu
    var = jnp.mean(xc * xc, axis=0)
    hn = xc * lax.rsqrt(var + EPS) * g0_ref[...] + bb0_ref[...]
    he = jnp.where(hn > 0, hn, jnp.exp(hn) - 1.0)
    xl1_ref[...] = jnp.dot(he, wl1_ref[...],
                           preferred_element_type=jnp.float32)
    xr1_ref[...] = jnp.dot(he, wr1_ref[...],
                           preferred_element_type=jnp.float32)


def _fin_body(o_ref, d_ref, r_ref, m_ref, b1_ref, g1_ref,
              bb1_ref, cw_ref, cb_ref, out_ref):
    raw = o_ref[0, pl.ds(0, N), :] + o_ref[1, pl.ds(0, N), :]
    den = d_ref[0, pl.ds(0, N), :] + d_ref[1, pl.ds(0, N), :]
    dexp = jnp.dot(den, r_ref[...], preferred_element_type=jnp.float32)
    hm = raw / (dexp + 1e-16)
    hv = jnp.dot(hm, m_ref[...], preferred_element_type=jnp.float32)
    hv = hv + b1_ref[...]
    mu = jnp.mean(hv, axis=0)
    xc = hv - mu
    var = jnp.mean(xc * xc, axis=0)
    hn = xc * lax.rsqrt(var + EPS) * g1_ref[...] + bb1_ref[...]
    out_ref[...] = jnp.dot(hn, cw_ref[...],
                           preferred_element_type=jnp.float32) + cb_ref[...]


@jax.jit
def kernel(x, edge_index, conv0_Wl, conv0_Wr, conv0_att, conv0_b, bn0_g,
           bn0_b, conv1_Wl, conv1_Wr, conv1_att, conv1_b, bn1_g, bn1_b,
           cls_W, cls_b):
    e = edge_index.shape[1]
    et = e + N
    blk = 2 * NW * B
    ep = ((et + blk - 1) // blk) * blk
    pad = ep - et

    ei = edge_index.astype(jnp.int32)
    loops = jnp.arange(N, dtype=jnp.int32)
    zpad = jnp.zeros((pad,), jnp.int32)
    src = jnp.concatenate([ei[0], loops, zpad])
    dstg = jnp.concatenate([ei[1], loops, zpad])
    dsts = jnp.concatenate([ei[1], loops, jnp.full((pad,), N, jnp.int32)])
    idxf = (dsts.reshape(-1, 1, B) * 8
            + jnp.arange(8, dtype=jnp.int32).reshape(1, 8, 1)).reshape(-1)

    mm2 = pl.pallas_call(
        _mm2_body,
        out_shape=[jax.ShapeDtypeStruct((N, F), jnp.float32)] * 2,
    )
    edge = _make_edge(ep)

    rmat = jnp.zeros((8, F), jnp.float32)
    rmat = rmat.at[jnp.repeat(jnp.arange(8), 16),
                   jnp.arange(F)].set(1.0)
    mmat = jnp.tile(jnp.eye(HID, dtype=jnp.float32), (HEADS, 1)) / HEADS

    xl0, xr0 = mm2(x, conv0_Wl, conv0_Wr)
    o0, den0 = edge(xl0, xr0, src, dstg, dsts, idxf, conv0_att.reshape(-1))

    mid = pl.pallas_call(
        _mid_body,
        out_shape=[jax.ShapeDtypeStruct((N, F), jnp.float32)] * 2,
    )
    xl1, xr1 = mid(o0, den0.reshape(NC, NP, 8),
                   rmat, conv0_b, bn0_g, bn0_b, conv1_Wl, conv1_Wr)

    o1, den1 = edge(xl1, xr1, src, dstg, dsts, idxf, conv1_att.reshape(-1))

    fin = pl.pallas_call(
        _fin_body,
        out_shape=jax.ShapeDtypeStruct((N, 2), jnp.float32),
    )
    return fin(o1, den1.reshape(NC, NP, 8),
               rmat, mmat, conv1_b, bn1_g, bn1_b, cls_W, cls_b)


# speedup vs baseline: 1.0472x; 1.0378x over previous
import jax
import jax.numpy as jnp
from jax import lax
from jax.experimental import pallas as pl
from jax.experimental.pallas import tpu as pltpu
from jax.experimental.pallas import tpu_sc as plsc

N = 10000
HID = 16
HEADS = 8
F = HEADS * HID
NEG = 0.2
EPS = 1e-5

NC = 2
NS = 16
NW = NC * NS
B = 96
NP = 10240
RPT = NP // NS

_mesh = plsc.VectorSubcoreMesh(core_axis_name="c", subcore_axis_name="s")
_SC_PARAMS = pltpu.CompilerParams(needs_layout_passes=False)


def _edge_body(xl, xr, src, dstg, dsts, idxf, attf,
               out, den,
               xlb, xrb, ob, exb, srcb, dgb, dsb, idxb, attb,
               out_sh, den_sh, sem0, sem1):
    c = lax.axis_index("c")
    s = lax.axis_index("s")
    wid = s * NC + c
    per_tile = src.shape[0] // NW
    n_chunks = per_tile // B
    zeros16 = jnp.zeros((16,), jnp.float32)
    lanes = lax.broadcasted_iota(jnp.int32, (16,), 0)

    pltpu.sync_copy(attf, attb.at[pl.ds(0, F)])
    for h in reversed(range(HEADS)):
        for j in reversed(range(HID)):
            cd = h * 16 + ((lanes + j) & 15)
            attb[pl.ds((h * 16 + j) * 16, 16)] = plsc.load_gather(
                attb.at[pl.ds(0, F)], [cd])

    @pl.loop(0, B // 2)
    def _(i):
        exb[pl.ds(i * 16, 16)] = zeros16

    @pl.loop(0, B)
    def _(i):
        for j in range(8):
            ob[i, pl.ds(j * 16, 16)] = zeros16

    d0 = s * RPT * 8
    for t in range((RPT + B - 1) // B):
        rem = min(B, RPT - t * B)
        pltpu.sync_copy(exb.at[pl.ds(0, rem * 8)],
                        den_sh.at[pl.ds(d0 + t * B * 8, rem * 8)])
    r0 = s * RPT
    for t in range((RPT + B - 1) // B):
        rem = min(B, RPT - t * B)
        pltpu.sync_copy(ob.at[pl.ds(0, rem)],
                        out_sh.at[pl.ds(r0 + t * B, rem)])

    base0 = wid * per_tile
    pltpu.sync_copy(src.at[pl.ds(base0, B)], srcb)
    pltpu.sync_copy(dstg.at[pl.ds(base0, B)], dgb)
    pltpu.sync_copy(dsts.at[pl.ds(base0, B)], dsb)
    pltpu.sync_copy(idxf.at[pl.ds(base0 * 8, B * 8)], idxb)
    pltpu.async_copy(xl.at[srcb], xlb, sem0)
    pltpu.async_copy(xr.at[dgb], xrb, sem1)
    plsc.subcore_barrier()

    @pl.loop(0, n_chunks)
    def _(k):
            pltpu.make_async_copy(xl.at[srcb], xlb, sem0).wait()
            pltpu.make_async_copy(xr.at[dgb], xrb, sem1).wait()

            @pl.loop(0, B // 16)
            def _(g):
                eidx = g * 16 + lanes

                @pl.loop(0, HEADS)
                def _(h):
                    acc = zeros16
                    for j in range(HID):
                        cd = h * 16 + ((lanes + j) & 15)
                        a = plsc.load_gather(xlb, [eidx, cd])
                        bv = plsc.load_gather(xrb, [eidx, cd])
                        m = a + bv
                        m = jnp.where(m > 0, m, NEG * m)
                        acc = acc + m * attb[pl.ds((h * 16 + j) * 16, 16)]
                    exv = jnp.exp(acc)
                    exb[pl.ds(h * B + g * 16, 16)] = exv
                    for j in range(HID):
                        cd = h * 16 + ((lanes + j) & 15)
                        a = plsc.load_gather(xlb, [eidx, cd])
                        plsc.store_scatter(ob, [eidx, cd], a * exv)

            pltpu.sync_copy(ob, out_sh.at[dsb], add=True)
            pltpu.sync_copy(exb, den_sh.at[idxb], add=True)

            @pl.when(k < n_chunks - 1)
            def _():
                base = wid * per_tile + (k + 1) * B
                pltpu.sync_copy(src.at[pl.ds(base, B)], srcb)
                pltpu.sync_copy(dstg.at[pl.ds(base, B)], dgb)
                pltpu.sync_copy(dsts.at[pl.ds(base, B)], dsb)
                pltpu.sync_copy(idxf.at[pl.ds(base * 8, B * 8)], idxb)
                pltpu.async_copy(xl.at[srcb], xlb, sem0)
                pltpu.async_copy(xr.at[dgb], xrb, sem1)

    plsc.subcore_barrier()
    for t in range((RPT + B - 1) // B):
        rem = min(B, RPT - t * B)
        pltpu.sync_copy(den_sh.at[pl.ds(d0 + t * B * 8, rem * 8)],
                        exb.at[pl.ds(0, rem * 8)])
        pltpu.sync_copy(exb.at[pl.ds(0, rem * 8)],
                        den.at[c, pl.ds(d0 + t * B * 8, rem * 8)])
    for t in range((RPT + B - 1) // B):
        rem = min(B, RPT - t * B)
        pltpu.sync_copy(out_sh.at[pl.ds(r0 + t * B, rem)],
                        xlb.at[pl.ds(0, rem)])
        pltpu.sync_copy(xlb.at[pl.ds(0, rem)],
                        out.at[c, pl.ds(r0 + t * B, rem)])


def _make_edge(ep):
    return pl.kernel(
        _edge_body,
        out_type=[
            jax.ShapeDtypeStruct((NC, NP, F), jnp.float32),
            jax.ShapeDtypeStruct((NC, NP * 8), jnp.float32),
        ],
        mesh=_mesh,
        compiler_params=_SC_PARAMS,
        scratch_types=[
            pltpu.VMEM((B, F), jnp.float32),
            pltpu.VMEM((B, F), jnp.float32),
            pltpu.VMEM((B, F), jnp.float32),
            pltpu.VMEM((B * 8,), jnp.float32),
            pltpu.VMEM((B,), jnp.int32),
            pltpu.VMEM((B,), jnp.int32),
            pltpu.VMEM((B,), jnp.int32),
            pltpu.VMEM((B * 8,), jnp.int32),
            pltpu.VMEM((F * 16,), jnp.float32),
            pltpu.VMEM_SHARED((NP, F), jnp.float32),
            pltpu.VMEM_SHARED((NP * 8,), jnp.float32),
            pltpu.SemaphoreType.DMA,
            pltpu.SemaphoreType.DMA,
        ],
    )


def _mm2_body(x_ref, wl_ref, wr_ref, xl_ref, xr_ref):
    x = x_ref[...]
    xl_ref[...] = jnp.dot(x, wl_ref[...], preferred_element_type=jnp.float32)
    xr_ref[...] = jnp.dot(x, wr_ref[...], preferred_element_type=jnp.float32)


def _mid_body(o_ref, d_ref, r_ref, b0_ref, g0_ref, bb0_ref,
              wl1_ref, wr1_ref, xl1_ref, xr1_ref):
    raw = o_ref[0, pl.ds(0, N), :] + o_ref[1, pl.ds(0, N), :]
    den = d_ref[0, pl.ds(0, N), :] + d_ref[1, pl.ds(0, N), :]
    dexp = jnp.dot(den, r_ref[...], preferred_element_type=jnp.float32)
    h = raw / (dexp + 1e-16) + b0_ref[...]
    mu = jnp.mean(h, axis=0)
    xc = h - mu
    var = jnp.mean(xc * xc, axis=0)
    hn = xc * lax.rsqrt(var + EPS) * g0_ref[...] + bb0_ref[...]
    he = jnp.where(hn > 0, hn, jnp.exp(hn) - 1.0)
    xl1_ref[...] = jnp.dot(he, wl1_ref[...],
                           preferred_element_type=jnp.float32)
    xr1_ref[...] = jnp.dot(he, wr1_ref[...],
                           preferred_element_type=jnp.float32)


def _fin_body(o_ref, d_ref, r_ref, m_ref, b1_ref, g1_ref,
              bb1_ref, cw_ref, cb_ref, out_ref):
    raw = o_ref[0, pl.ds(0, N), :] + o_ref[1, pl.ds(0, N), :]
    den = d_ref[0, pl.ds(0, N), :] + d_ref[1, pl.ds(0, N), :]
    dexp = jnp.dot(den, r_ref[...], preferred_element_type=jnp.float32)
    hm = raw / (dexp + 1e-16)
    hv = jnp.dot(hm, m_ref[...], preferred_element_type=jnp.float32)
    hv = hv + b1_ref[...]
    mu = jnp.mean(hv, axis=0)
    xc = hv - mu
    var = jnp.mean(xc * xc, axis=0)
    hn = xc * lax.rsqrt(var + EPS) * g1_ref[...] + bb1_ref[...]
    out_ref[...] = jnp.dot(hn, cw_ref[...],
                           preferred_element_type=jnp.float32) + cb_ref[...]


@jax.jit
def kernel(x, edge_index, conv0_Wl, conv0_Wr, conv0_att, conv0_b, bn0_g,
           bn0_b, conv1_Wl, conv1_Wr, conv1_att, conv1_b, bn1_g, bn1_b,
           cls_W, cls_b):
    e = edge_index.shape[1]
    et = e + N
    blk = 2 * NW * B
    ep = ((et + blk - 1) // blk) * blk
    pad = ep - et

    ei = edge_index.astype(jnp.int32)
    loops = jnp.arange(N, dtype=jnp.int32)
    zpad = jnp.zeros((pad,), jnp.int32)
    src = jnp.concatenate([ei[0], loops, zpad])
    dstg = jnp.concatenate([ei[1], loops, zpad])
    dsts = jnp.concatenate([ei[1], loops, jnp.full((pad,), N, jnp.int32)])
    idxf = (dsts.reshape(-1, 1, B) * 8
            + jnp.arange(8, dtype=jnp.int32).reshape(1, 8, 1)).reshape(-1)

    mm2 = pl.pallas_call(
        _mm2_body,
        out_shape=[jax.ShapeDtypeStruct((N, F), jnp.float32)] * 2,
    )
    edge = _make_edge(ep)

    rmat = jnp.zeros((8, F), jnp.float32)
    rmat = rmat.at[jnp.repeat(jnp.arange(8), 16),
                   jnp.arange(F)].set(1.0)
    mmat = jnp.tile(jnp.eye(HID, dtype=jnp.float32), (HEADS, 1)) / HEADS

    xl0, xr0 = mm2(x, conv0_Wl, conv0_Wr)
    o0, den0 = edge(xl0, xr0, src, dstg, dsts, idxf, conv0_att.reshape(-1))

    mid = pl.pallas_call(
        _mid_body,
        out_shape=[jax.ShapeDtypeStruct((N, F), jnp.float32)] * 2,
    )
    xl1, xr1 = mid(o0, den0.reshape(NC, NP, 8),
                   rmat, conv0_b, bn0_g, bn0_b, conv1_Wl, conv1_Wr)

    o1, den1 = edge(xl1, xr1, src, dstg, dsts, idxf, conv1_att.reshape(-1))

    fin = pl.pallas_call(
        _fin_body,
        out_shape=jax.ShapeDtypeStruct((N, 2), jnp.float32),
    )
    return fin(o1, den1.reshape(NC, NP, 8),
               rmat, mmat, conv1_b, bn1_g, bn1_b, cls_W, cls_b)
